# baseline (speedup 1.0000x reference)
import jax
import jax.numpy as jnp
from jax.experimental import pallas as pl
from jax.experimental.pallas import tpu as pltpu
from jax.experimental.pallas import tpu_sc as plsc


def _sc_gather_pairs(table2, ids2):
    batch = ids2.shape[0]
    row = table2.shape[1]
    window = 128
    indices = ids2.reshape(1, batch)
    mesh = plsc.VectorSubcoreMesh(core_axis_name="core", subcore_axis_name="subcore")

    @pl.kernel(
        out_type=jax.ShapeDtypeStruct((batch, row), table2.dtype),
        mesh=mesh,
    )
    def gather_kernel(tbl_hbm, idx_hbm, out_hbm):
        def body(idx_vmem, out_vmem):
            pltpu.sync_copy(tbl_hbm.at[idx_vmem.at[0]], out_vmem)

        pltpu.emit_pipeline(
            body,
            grid=(batch // window,),
            in_specs=[pl.BlockSpec((1, window), index_map=lambda i: (0, i))],
            out_specs=[pl.BlockSpec((window, row), index_map=lambda i: (i, 0))],
            core_axis_name="subcore",
            dimension_semantics=(pltpu.PARALLEL,),
        )(idx_hbm, out_hbm)

    return gather_kernel(table2, indices)


_VT = 2048
_NO = 4
_NW = 3
_NC = 4


def _tc_head(x_pair, parity, head_w):
    b = x_pair.shape[0]
    v, d = head_w.shape
    vt = _VT
    steps = (v + vt - 1) // vt

    tail = v - (steps - 1) * vt
    tail_hi = tail % 128
    tail_lo = tail - tail_hi

    def mm_kernel(
        xp_ref, par_ref, w_hbm, o_hbm, xs_ref, obuf, wbuf, otail, osem, wsem, tsem
    ):
        def wcopy(i):
            wi = min(vt, v - i * vt)
            return pltpu.make_async_copy(
                w_hbm.at[pl.ds(i * vt, wi), :],
                wbuf.at[i % _NW, pl.ds(0, wi), :],
                wsem.at[i % _NW],
            )

        def ochunks(i):
            wi = vt if i < steps - 1 else tail_lo
            copies = []
            for c in range(_NC):
                c0 = c * wi // _NC
                c1 = (c + 1) * wi // _NC
                c0, c1 = (c0 // 128) * 128, (c1 // 128) * 128
                if c1 > c0:
                    copies.append(
                        pltpu.make_async_copy(
                            obuf.at[i % _NO, :, pl.ds(c0, c1 - c0)],
                            o_hbm.at[:, pl.ds(i * vt + c0, c1 - c0)],
                            osem.at[i % _NO, c],
                        )
                    )
            return copies

        def ostart(i):
            for cp in ochunks(i):
                cp.start()

        def owait(i):
            for cp in ochunks(i):
                cp.wait()

        def tcopy():
            return pltpu.make_async_copy(
                otail,
                o_hbm.at[:, pl.ds(v - tail_hi, tail_hi)],
                tsem,
            )

        par = par_ref[...]
        xs_ref[...] = (
            xp_ref[:, :d] * (1.0 - par) + xp_ref[:, d:] * par
        ).astype(jnp.bfloat16)

        for j in range(min(_NW, steps)):
            wcopy(j).start()
        for i in range(steps):
            wcopy(i).wait()
            if i >= _NO:
                owait(i - _NO)
            obuf[i % _NO] = jax.lax.dot_general(
                xs_ref[...],
                wbuf[i % _NW].astype(jnp.bfloat16),
                dimension_numbers=(((1,), (1,)), ((), ())),
                preferred_element_type=jnp.float32,
            )
            if i == steps - 1 and tail_hi:
                otail[...] = jax.lax.dot_general(
                    xs_ref[...],
                    wbuf[i % _NW, pl.ds(tail_lo, tail_hi), :].astype(jnp.bfloat16),
                    dimension_numbers=(((1,), (1,)), ((), ())),
                    preferred_element_type=jnp.float32,
                )
                tcopy().start()
            ostart(i)
            if i + _NW < steps:
                wcopy(i + _NW).start()
        for i in range(max(0, steps - _NO), steps):
            owait(i)
        if tail_hi:
            tcopy().wait()

    return pl.pallas_call(
        mm_kernel,
        grid=(1,),
        in_specs=[
            pl.BlockSpec((b, 2 * d), lambda i: (0, 0)),
            pl.BlockSpec((b, 1), lambda i: (0, 0)),
            pl.BlockSpec(memory_space=pl.ANY),
        ],
        out_specs=pl.BlockSpec(memory_space=pl.ANY),
        out_shape=jax.ShapeDtypeStruct((b, v), jnp.float32),
        scratch_shapes=[
            pltpu.VMEM((b, d), jnp.bfloat16),
            pltpu.VMEM((_NO, b, vt), jnp.float32),
            pltpu.VMEM((_NW, vt, d), jnp.float32),
            pltpu.VMEM((b, 32), jnp.float32),
            pltpu.SemaphoreType.DMA((_NO, _NC)),
            pltpu.SemaphoreType.DMA((_NW,)),
            pltpu.SemaphoreType.DMA,
        ],
    )(x_pair, parity, head_w)


def kernel(input_ids, embed_table, head_w):
    v, d = embed_table.shape
    table2 = embed_table.reshape(v // 2, 2 * d)
    ids2 = jax.lax.shift_right_logical(input_ids, 1)
    parity = (input_ids & 1).astype(jnp.float32).reshape(-1, 1)
    x_pair = _sc_gather_pairs(table2, ids2)
    return _tc_head(x_pair, parity, head_w)


# speedup vs baseline: 2.7932x; 2.7932x over previous
import jax
import jax.numpy as jnp
from jax.experimental import pallas as pl
from jax.experimental.pallas import tpu as pltpu
from jax.experimental.pallas import tpu_sc as plsc


def _sc_gather_pairs(table2, ids2):
    batch = ids2.shape[0]
    row = table2.shape[1]
    window = 128
    indices = ids2.reshape(1, batch)
    mesh = plsc.VectorSubcoreMesh(core_axis_name="core", subcore_axis_name="subcore")

    @pl.kernel(
        out_type=jax.ShapeDtypeStruct((batch, row), table2.dtype),
        mesh=mesh,
    )
    def gather_kernel(tbl_hbm, idx_hbm, out_hbm):
        def body(idx_vmem, out_vmem):
            pltpu.sync_copy(tbl_hbm.at[idx_vmem.at[0]], out_vmem)

        pltpu.emit_pipeline(
            body,
            grid=(batch // window,),
            in_specs=[pl.BlockSpec((1, window), index_map=lambda i: (0, i))],
            out_specs=[pl.BlockSpec((window, row), index_map=lambda i: (i, 0))],
            core_axis_name="subcore",
            dimension_semantics=(pltpu.PARALLEL,),
        )(idx_hbm, out_hbm)

    return gather_kernel(table2, indices)


_VT = 2048
_NO = 4
_NW = 3
_NC = 4


def _tc_head_t(x_pair, parity, w_t):
    b = x_pair.shape[0]
    d, v = w_t.shape
    vt = _VT
    steps = (v + vt - 1) // vt
    tail = v - (steps - 1) * vt
    tail_hi = tail % 128
    tail_lo = tail - tail_hi

    def mm_kernel(
        xp_ref, par_ref, w_hbm, o_hbm, xs_ref, obuf, wbuf, wtl, osem, wsem, tsem
    ):
        def wcopies(i):
            if i < steps - 1:
                return [
                    pltpu.make_async_copy(
                        w_hbm.at[:, pl.ds(i * vt, vt)],
                        wbuf.at[i % _NW],
                        wsem.at[i % _NW],
                    )
                ]
            cps = [
                pltpu.make_async_copy(
                    w_hbm.at[:, pl.ds(i * vt, tail_lo)],
                    wbuf.at[i % _NW, :, pl.ds(0, tail_lo)],
                    wsem.at[i % _NW],
                )
            ]
            if tail_hi:
                cps.append(
                    pltpu.make_async_copy(
                        w_hbm.at[:, pl.ds(i * vt + tail_lo, tail_hi)],
                        wtl,
                        tsem,
                    )
                )
            return cps

        def ochunks(i):
            rows = vt if i < steps - 1 else tail
            copies = []
            for c in range(_NC):
                r0 = (c * rows // _NC) // 8 * 8
                r1 = ((c + 1) * rows // _NC) // 8 * 8 if c < _NC - 1 else rows
                if r1 > r0:
                    copies.append(
                        pltpu.make_async_copy(
                            obuf.at[i % _NO, pl.ds(r0, r1 - r0), :],
                            o_hbm.at[pl.ds(i * vt + r0, r1 - r0), :],
                            osem.at[i % _NO, c],
                        )
                    )
            return copies

        def ostart(i):
            for cp in ochunks(i):
                cp.start()

        def owait(i):
            for cp in ochunks(i):
                cp.wait()

        par = par_ref[...]
        xs_ref[...] = (
            xp_ref[:, :d] * (1.0 - par) + xp_ref[:, d:] * par
        ).astype(jnp.bfloat16)

        for j in range(min(_NW, steps)):
            for cp in wcopies(j):
                cp.start()
        for i in range(steps):
            for cp in wcopies(i):
                cp.wait()
            if i >= _NO:
                owait(i - _NO)
            if i < steps - 1:
                obuf[i % _NO] = jax.lax.dot_general(
                    wbuf[i % _NW].astype(jnp.bfloat16),
                    xs_ref[...],
                    dimension_numbers=(((0,), (1,)), ((), ())),
                    preferred_element_type=jnp.float32,
                )
            else:
                obuf[i % _NO, pl.ds(0, tail_lo), :] = jax.lax.dot_general(
                    wbuf[i % _NW, :, pl.ds(0, tail_lo)].astype(jnp.bfloat16),
                    xs_ref[...],
                    dimension_numbers=(((0,), (1,)), ((), ())),
                    preferred_element_type=jnp.float32,
                )
                if tail_hi:
                    obuf[i % _NO, pl.ds(tail_lo, tail_hi), :] = jax.lax.dot_general(
                        wtl[...].astype(jnp.bfloat16),
                        xs_ref[...],
                        dimension_numbers=(((0,), (1,)), ((), ())),
                        preferred_element_type=jnp.float32,
                    )
            ostart(i)
            if i + _NW < steps:
                for cp in wcopies(i + _NW):
                    cp.start()
        for i in range(max(0, steps - _NO), steps):
            owait(i)

    return pl.pallas_call(
        mm_kernel,
        grid=(1,),
        in_specs=[
            pl.BlockSpec((b, 2 * d), lambda i: (0, 0)),
            pl.BlockSpec((b, 1), lambda i: (0, 0)),
            pl.BlockSpec(memory_space=pl.ANY),
        ],
        out_specs=pl.BlockSpec(memory_space=pl.ANY),
        out_shape=jax.ShapeDtypeStruct((v, b), jnp.float32),
        scratch_shapes=[
            pltpu.VMEM((b, d), jnp.bfloat16),
            pltpu.VMEM((_NO, vt, b), jnp.float32),
            pltpu.VMEM((_NW, d, vt), jnp.float32),
            pltpu.VMEM((d, 32), jnp.float32),
            pltpu.SemaphoreType.DMA((_NO, _NC)),
            pltpu.SemaphoreType.DMA((_NW,)),
            pltpu.SemaphoreType.DMA,
        ],
    )(x_pair, parity, w_t)


def kernel(input_ids, embed_table, head_w):
    v, d = embed_table.shape
    table2 = embed_table.reshape(v // 2, 2 * d)
    ids2 = jax.lax.shift_right_logical(input_ids, 1)
    parity = (input_ids & 1).astype(jnp.float32).reshape(-1, 1)
    x_pair = _sc_gather_pairs(table2, ids2)
    out_t = _tc_head_t(x_pair, parity, head_w.T)
    return out_t.T
